# baseline (speedup 1.0000x reference)
import functools

import jax
import jax.numpy as jnp
from jax import lax
from jax.experimental import pallas as pl
from jax.experimental.pallas import tpu as pltpu
from jax.experimental.pallas import tpu_sc as plsc

_N = 10000
_M = 5000
_P = 320000
_C1 = 128
_HEADS = 4
_CH = 32
_NCLS = 40
_NEG = 0.2

_NC = 2
_NS = 16
_NW = _NC * _NS
_K1, _GRP1, _NGRP1 = 100, 20, 5
_K2, _GRP2, _NGRP2 = 125, 80, 1

_MP = 5120
_NP = 10240
_B = 512

_D1 = 144
_D2 = 48


def _seg_sum_sc(table, gidx, sidx, n_dst_pad, D, K, grp, ngrp):
    rpt = n_dst_pad // _NS
    mesh = plsc.VectorSubcoreMesh(core_axis_name="c", subcore_axis_name="s",
                                  num_cores=_NC, num_subcores=_NS)

    def body(table_hbm, gidx_hbm, sidx_hbm, out_hbm, gv, sv,
             buf_a, buf_b, zb, acc, sem_a, sem_b, sem_sa, sem_sb):
        c = lax.axis_index("c")
        s = lax.axis_index("s")
        wid = c * _NS + s

        def zrow(r, carry):
            for cc in range(D // 16):
                zb[r, pl.ds(cc * 16, 16)] = jnp.zeros((16,), jnp.float32)
            return carry
        lax.fori_loop(0, 16, zrow, 0)

        def zacc(k, carry):
            pltpu.sync_copy(zb, acc.at[pl.ds(s * rpt + k * 16, 16)])
            return carry
        lax.fori_loop(0, rpt // 16, zacc, 0)
        plsc.subcore_barrier()

        def gstart(j, buf, sem):
            pltpu.async_copy(table_hbm.at[gv.at[j]], buf, sem)

        def gwait(j, buf, sem):
            pltpu.make_async_copy(table_hbm.at[gv.at[j]], buf, sem).wait()

        def group(g, carry):
            pltpu.sync_copy(gidx_hbm.at[wid, g], gv)
            pltpu.sync_copy(sidx_hbm.at[wid, g], sv)
            gstart(0, buf_a, sem_a)
            gstart(1, buf_b, sem_b)

            def step(i, carry2):
                j = i * 2
                gwait(j, buf_a, sem_a)
                d_a = pltpu.async_copy(buf_a, acc.at[sv.at[j]], sem_sa,
                                       add=True)
                gwait(j + 1, buf_b, sem_b)
                d_b = pltpu.async_copy(buf_b, acc.at[sv.at[j + 1]], sem_sb,
                                       add=True)
                d_a.wait()

                @pl.when(i < grp // 2 - 1)
                def _():
                    gstart(j + 2, buf_a, sem_a)
                d_b.wait()

                @pl.when(i < grp // 2 - 1)
                def _():
                    gstart(j + 3, buf_b, sem_b)
                return carry2
            lax.fori_loop(0, grp // 2, step, 0)
            return carry
        lax.fori_loop(0, ngrp, group, 0)
        plsc.subcore_barrier()

        pltpu.sync_copy(acc.at[pl.ds(s * rpt, rpt)],
                        out_hbm.at[pl.ds(c * n_dst_pad + s * rpt, rpt)])

    fn = pl.kernel(
        body,
        out_type=jax.ShapeDtypeStruct((_NC * n_dst_pad, D), jnp.float32),
        mesh=mesh,
        scratch_types=[
            pltpu.VMEM((grp, K), jnp.int32),
            pltpu.VMEM((grp, K), jnp.int32),
            pltpu.VMEM((K, D), jnp.float32),
            pltpu.VMEM((K, D), jnp.float32),
            pltpu.VMEM((16, D), jnp.float32),
            pltpu.VMEM_SHARED((n_dst_pad, D), jnp.float32),
            pltpu.SemaphoreType.DMA,
            pltpu.SemaphoreType.DMA,
            pltpu.SemaphoreType.DMA,
            pltpu.SemaphoreType.DMA,
        ],
        compiler_params=pltpu.CompilerParams(use_tc_tiling_on_sc=False),
    )
    return fn(table, gidx, sidx)


def _lrelu(x):
    return jnp.where(x >= 0, x, _NEG * x)


def _elu(x):
    return jnp.where(x > 0, x, jnp.exp(jnp.minimum(x, 0.0)) - 1.0)


def _xform1(X_pad, Wc, bc):
    def body(x_ref, w_ref, b_ref, o_ref):
        xt = jnp.dot(x_ref[...], w_ref[...],
                     preferred_element_type=jnp.float32) + b_ref[...][None, :]
        o_ref[:, :_C1] = xt
        ii = lax.broadcasted_iota(jnp.int32, (_B, _D1 - _C1), 1)
        o_ref[:, _C1:] = jnp.where(ii == 0, 1.0, 0.0)

    return pl.pallas_call(
        body,
        grid=(_NP // _B,),
        in_specs=[
            pl.BlockSpec((_B, _C1), lambda j: (j, 0)),
            pl.BlockSpec((_C1, _C1), lambda j: (0, 0)),
            pl.BlockSpec((_C1,), lambda j: (0,)),
        ],
        out_specs=pl.BlockSpec((_B, _D1), lambda j: (j, 0)),
        out_shape=jax.ShapeDtypeStruct((_NP, _D1), jnp.float32),
    )(X_pad, Wc, bc)


def _edge_stage1(P1, ae):
    def body(p0_ref, p1_ref, ae_ref, y_ref, a_ref, d_ref):
        p = p0_ref[...] + p1_ref[...]
        deg = p[:, _C1]
        degc = jnp.clip(deg, 1.0, None)
        d_ref[:, 0] = degc
        y = p[:, :_C1] / degc[:, None]
        y_ref[...] = y
        for h in range(_HEADS):
            blk = y[:, _CH * h:_CH * (h + 1)]
            a_ref[:, h] = jnp.sum(blk * ae_ref[h][None, :], axis=1)

    nb = _MP // _B
    return pl.pallas_call(
        body,
        grid=(nb,),
        in_specs=[
            pl.BlockSpec((_B, _D1), lambda j: (j, 0)),
            pl.BlockSpec((_B, _D1), lambda j, nb=nb: (nb + j, 0)),
            pl.BlockSpec((_HEADS, _CH), lambda j: (0, 0)),
        ],
        out_specs=[
            pl.BlockSpec((_B, _C1), lambda j: (j, 0)),
            pl.BlockSpec((_B, _HEADS), lambda j: (j, 0)),
            pl.BlockSpec((_B, 1), lambda j: (j, 0)),
        ],
        out_shape=[
            jax.ShapeDtypeStruct((_MP, _C1), jnp.float32),
            jax.ShapeDtypeStruct((_MP, _HEADS), jnp.float32),
            jax.ShapeDtypeStruct((_MP, 1), jnp.float32),
        ],
    )(P1, P1, ae)


def _scale_rows1(Y, alpha):
    def body(y_ref, af_ref, ab_ref, out_ref):
        y = y_ref[...]
        m = jnp.max(_lrelu(af_ref[...]), axis=0)
        om_blk = jnp.exp(_lrelu(ab_ref[...]) - m[None, :])
        for h in range(_HEADS):
            out_ref[:, _CH * h:_CH * (h + 1)] = (
                y[:, _CH * h:_CH * (h + 1)] * om_blk[:, h][:, None])
        ii = lax.broadcasted_iota(jnp.int32, (_B, _D1 - _C1), 1)
        pad = jnp.zeros((_B, _D1 - _C1), jnp.float32)
        for h in range(_HEADS):
            pad = jnp.where(ii == h, om_blk[:, h][:, None], pad)
        out_ref[:, _C1:] = pad

    return pl.pallas_call(
        body,
        grid=(_MP // _B,),
        in_specs=[
            pl.BlockSpec((_B, _C1), lambda j: (j, 0)),
            pl.BlockSpec((_MP, _HEADS), lambda j: (0, 0)),
            pl.BlockSpec((_B, _HEADS), lambda j: (j, 0)),
        ],
        out_specs=pl.BlockSpec((_B, _D1), lambda j: (j, 0)),
        out_shape=jax.ShapeDtypeStruct((_MP, _D1), jnp.float32),
    )(Y, alpha, alpha)


def _xform2(P2, Wo, bo):
    def body(p0_ref, p1_ref, w_ref, b_ref, o_ref):
        p = p0_ref[...] + p1_ref[...]
        cols = []
        for h in range(_HEADS):
            s = p[:, _C1 + h]
            u = p[:, _CH * h:_CH * (h + 1)]
            cols.append(_elu(u / (s + 1e-12)[:, None]))
        hfeat = jnp.concatenate(cols, axis=1)
        zt = jnp.dot(hfeat, w_ref[...],
                     preferred_element_type=jnp.float32) + b_ref[...][None, :]
        o_ref[:, :_NCLS] = zt
        o_ref[:, _NCLS:] = jnp.zeros((_B, _D2 - _NCLS), jnp.float32)

    nb = _NP // _B
    return pl.pallas_call(
        body,
        grid=(nb,),
        in_specs=[
            pl.BlockSpec((_B, _D1), lambda j: (j, 0)),
            pl.BlockSpec((_B, _D1), lambda j, nb=nb: (nb + j, 0)),
            pl.BlockSpec((_C1, _NCLS), lambda j: (0, 0)),
            pl.BlockSpec((_NCLS,), lambda j: (0,)),
        ],
        out_specs=pl.BlockSpec((_B, _D2), lambda j: (j, 0)),
        out_shape=jax.ShapeDtypeStruct((_NP, _D2), jnp.float32),
    )(P2, P2, Wo, bo)


def _edge_stage2(P3, degc, aeo):
    def body(p0_ref, p1_ref, d_ref, ae_ref, y_ref, a_ref):
        p = p0_ref[...] + p1_ref[...]
        y = p[:, :_NCLS] / d_ref[...]
        y_ref[...] = y
        a_ref[:, 0] = jnp.sum(y * ae_ref[...][None, :], axis=1)

    nb = _MP // _B
    return pl.pallas_call(
        body,
        grid=(nb,),
        in_specs=[
            pl.BlockSpec((_B, _D2), lambda j: (j, 0)),
            pl.BlockSpec((_B, _D2), lambda j, nb=nb: (nb + j, 0)),
            pl.BlockSpec((_B, 1), lambda j: (j, 0)),
            pl.BlockSpec((_NCLS,), lambda j: (0,)),
        ],
        out_specs=[
            pl.BlockSpec((_B, _NCLS), lambda j: (j, 0)),
            pl.BlockSpec((_B, 1), lambda j: (j, 0)),
        ],
        out_shape=[
            jax.ShapeDtypeStruct((_MP, _NCLS), jnp.float32),
            jax.ShapeDtypeStruct((_MP, 1), jnp.float32),
        ],
    )(P3, P3, degc, aeo)


def _scale_rows2(Y2, alpha2):
    def body(y_ref, af_ref, ab_ref, out_ref):
        m = jnp.max(_lrelu(af_ref[...]), axis=0)
        om_col = jnp.exp(_lrelu(ab_ref[...]) - m[None, :])
        out_ref[:, :_NCLS] = y_ref[...] * om_col
        ii = lax.broadcasted_iota(jnp.int32, (_B, _D2 - _NCLS), 1)
        out_ref[:, _NCLS:] = jnp.where(ii == 0, om_col, 0.0)

    return pl.pallas_call(
        body,
        grid=(_MP // _B,),
        in_specs=[
            pl.BlockSpec((_B, _NCLS), lambda j: (j, 0)),
            pl.BlockSpec((_MP, 1), lambda j: (0, 0)),
            pl.BlockSpec((_B, 1), lambda j: (j, 0)),
        ],
        out_specs=pl.BlockSpec((_B, _D2), lambda j: (j, 0)),
        out_shape=jax.ShapeDtypeStruct((_MP, _D2), jnp.float32),
    )(Y2, alpha2, alpha2)


def _finalize(P4):
    def body(p0_ref, p1_ref, o_ref):
        p = p0_ref[...] + p1_ref[...]
        s = p[:, _NCLS]
        o_ref[...] = _elu(p[:, :_NCLS] / (s + 1e-12)[:, None])

    nb = _NP // _B
    return pl.pallas_call(
        body,
        grid=(nb,),
        in_specs=[
            pl.BlockSpec((_B, _D2), lambda j: (j, 0)),
            pl.BlockSpec((_B, _D2), lambda j, nb=nb: (nb + j, 0)),
        ],
        out_specs=pl.BlockSpec((_B, _NCLS), lambda j: (j, 0)),
        out_shape=jax.ShapeDtypeStruct((_NP, _NCLS), jnp.float32),
    )(P4, P4)


def kernel(X, W, b, ae, Wo, bo, aeo, v_ids, e_ids):
    Wc = W.transpose(1, 0, 2).reshape(_C1, _C1)
    bc = b.reshape(_C1)
    v1 = v_ids.reshape(_NW, _NGRP1, _GRP1, _K1)
    e1 = e_ids.reshape(_NW, _NGRP1, _GRP1, _K1)
    v2 = v_ids.reshape(_NW, _NGRP2, _GRP2, _K2)
    e2 = e_ids.reshape(_NW, _NGRP2, _GRP2, _K2)
    X_pad = jnp.pad(X, ((0, _NP - _N), (0, 0)))

    Xt_aug = _xform1(X_pad, Wc, bc)
    P1 = _seg_sum_sc(Xt_aug, v1, e1, _MP, _D1, _K1, _GRP1, _NGRP1)
    Y, alpha, degc = _edge_stage1(P1, ae)
    Yhat1 = _scale_rows1(Y, alpha)
    P2 = _seg_sum_sc(Yhat1, e1, v1, _NP, _D1, _K1, _GRP1, _NGRP1)

    Zt2_aug = _xform2(P2, Wo, bo)
    P3 = _seg_sum_sc(Zt2_aug, v2, e2, _MP, _D2, _K2, _GRP2, _NGRP2)
    Y2, alpha2 = _edge_stage2(P3, degc, aeo)
    Yhat2 = _scale_rows2(Y2, alpha2)
    P4 = _seg_sum_sc(Yhat2, e2, v2, _NP, _D2, _K2, _GRP2, _NGRP2)

    out = _finalize(P4)
    return out[:_N]


# speedup vs baseline: 1.1598x; 1.1598x over previous
import functools

import jax
import jax.numpy as jnp
from jax import lax
from jax.experimental import pallas as pl
from jax.experimental.pallas import tpu as pltpu
from jax.experimental.pallas import tpu_sc as plsc

_N = 10000
_M = 5000
_P = 320000
_C1 = 128
_HEADS = 4
_CH = 32
_NCLS = 40
_NEG = 0.2

_NC = 2
_NS = 16
_NW = _NC * _NS
_K1, _GRP1, _NGRP1 = 100, 20, 5
_K2, _GRP2, _NGRP2 = 125, 80, 1

_MP = 5120
_NP = 10240
_B = 512

_D1 = 144
_D2 = 48


def _seg_sum_sc(table, gidx, sidx, n_dst_pad, D, K, grp, ngrp):
    rpt = n_dst_pad // _NS
    mesh = plsc.VectorSubcoreMesh(core_axis_name="c", subcore_axis_name="s",
                                  num_cores=_NC, num_subcores=_NS)

    def body(table_hbm, gidx_hbm, sidx_hbm, out_hbm, gv, sv,
             buf_a, buf_b, zb, acc, sem_a, sem_b):
        c = lax.axis_index("c")
        s = lax.axis_index("s")
        wid = c * _NS + s

        def zrow(r, carry):
            for cc in range(D // 16):
                zb[r, pl.ds(cc * 16, 16)] = jnp.zeros((16,), jnp.float32)
            return carry
        lax.fori_loop(0, 16, zrow, 0)

        def zacc(k, carry):
            pltpu.sync_copy(zb, acc.at[pl.ds(s * rpt + k * 16, 16)])
            return carry
        lax.fori_loop(0, rpt // 16, zacc, 0)
        plsc.subcore_barrier()

        def gstart(j, buf, sem):
            pltpu.async_copy(table_hbm.at[gv.at[j]], buf, sem)

        def gwait(j, buf, sem):
            pltpu.make_async_copy(table_hbm.at[gv.at[j]], buf, sem).wait()

        def group(g, carry):
            pltpu.sync_copy(gidx_hbm.at[wid, g], gv)
            pltpu.sync_copy(sidx_hbm.at[wid, g], sv)
            gstart(0, buf_a, sem_a)

            def step(i, carry2):
                j = i * 2
                gstart(j + 1, buf_b, sem_b)
                gwait(j, buf_a, sem_a)
                pltpu.sync_copy(buf_a, acc.at[sv.at[j]], add=True)

                @pl.when(i < grp // 2 - 1)
                def _():
                    gstart(j + 2, buf_a, sem_a)

                gwait(j + 1, buf_b, sem_b)
                pltpu.sync_copy(buf_b, acc.at[sv.at[j + 1]], add=True)
                return carry2
            lax.fori_loop(0, grp // 2, step, 0)
            return carry
        lax.fori_loop(0, ngrp, group, 0)
        plsc.subcore_barrier()

        pltpu.sync_copy(acc.at[pl.ds(s * rpt, rpt)],
                        out_hbm.at[pl.ds(c * n_dst_pad + s * rpt, rpt)])

    fn = pl.kernel(
        body,
        out_type=jax.ShapeDtypeStruct((_NC * n_dst_pad, D), jnp.float32),
        mesh=mesh,
        scratch_types=[
            pltpu.VMEM((grp, K), jnp.int32),
            pltpu.VMEM((grp, K), jnp.int32),
            pltpu.VMEM((K, D), jnp.float32),
            pltpu.VMEM((K, D), jnp.float32),
            pltpu.VMEM((16, D), jnp.float32),
            pltpu.VMEM_SHARED((n_dst_pad, D), jnp.float32),
            pltpu.SemaphoreType.DMA,
            pltpu.SemaphoreType.DMA,
        ],
        compiler_params=pltpu.CompilerParams(use_tc_tiling_on_sc=False),
    )
    return fn(table, gidx, sidx)


def _lrelu(x):
    return jnp.where(x >= 0, x, _NEG * x)


def _elu(x):
    return jnp.where(x > 0, x, jnp.exp(jnp.minimum(x, 0.0)) - 1.0)


def _xform1(X_pad, Wc, bc):
    def body(x_ref, w_ref, b_ref, o_ref):
        xt = jnp.dot(x_ref[...], w_ref[...],
                     preferred_element_type=jnp.float32) + b_ref[...][None, :]
        o_ref[:, :_C1] = xt
        ii = lax.broadcasted_iota(jnp.int32, (_B, _D1 - _C1), 1)
        o_ref[:, _C1:] = jnp.where(ii == 0, 1.0, 0.0)

    return pl.pallas_call(
        body,
        grid=(_NP // _B,),
        in_specs=[
            pl.BlockSpec((_B, _C1), lambda j: (j, 0)),
            pl.BlockSpec((_C1, _C1), lambda j: (0, 0)),
            pl.BlockSpec((_C1,), lambda j: (0,)),
        ],
        out_specs=pl.BlockSpec((_B, _D1), lambda j: (j, 0)),
        out_shape=jax.ShapeDtypeStruct((_NP, _D1), jnp.float32),
    )(X_pad, Wc, bc)


def _edge_combined1(P1, ae):
    nb = _MP // _B

    def body(p0_ref, p1_ref, ae_ref, yhat_ref, d_ref, m_ref):
        p_idx = pl.program_id(0)
        j = pl.program_id(1)

        @pl.when(jnp.logical_and(p_idx == 0, j == 0))
        def _():
            m_ref[...] = jnp.zeros((1, _HEADS), jnp.float32)

        p = p0_ref[...] + p1_ref[...]
        degc = jnp.clip(p[:, _C1], 1.0, None)
        d_ref[:, 0] = degc
        y = p[:, :_C1] / degc[:, None]
        sc_cols = []
        for h in range(_HEADS):
            blk = y[:, _CH * h:_CH * (h + 1)]
            a_h = jnp.sum(blk * ae_ref[h][None, :], axis=1)
            sc_cols.append(_lrelu(a_h)[:, None])
        sc = jnp.concatenate(sc_cols, axis=1)
        m_ref[...] = jnp.maximum(m_ref[...], jnp.max(sc, axis=0)[None, :])
        om_blk = jnp.exp(sc - m_ref[...])
        for h in range(_HEADS):
            yhat_ref[:, _CH * h:_CH * (h + 1)] = (
                y[:, _CH * h:_CH * (h + 1)] * om_blk[:, h][:, None])
        ii = lax.broadcasted_iota(jnp.int32, (_B, _D1 - _C1), 1)
        pad = jnp.zeros((_B, _D1 - _C1), jnp.float32)
        for h in range(_HEADS):
            pad = jnp.where(ii == h, om_blk[:, h][:, None], pad)
        yhat_ref[:, _C1:] = pad

    return pl.pallas_call(
        body,
        grid=(2, nb),
        in_specs=[
            pl.BlockSpec((_B, _D1), lambda p, j: (j, 0)),
            pl.BlockSpec((_B, _D1), lambda p, j, nb=nb: (nb + j, 0)),
            pl.BlockSpec((_HEADS, _CH), lambda p, j: (0, 0)),
        ],
        out_specs=[
            pl.BlockSpec((_B, _D1), lambda p, j: (j, 0)),
            pl.BlockSpec((_B, 1), lambda p, j: (j, 0)),
        ],
        out_shape=[
            jax.ShapeDtypeStruct((_MP, _D1), jnp.float32),
            jax.ShapeDtypeStruct((_MP, 1), jnp.float32),
        ],
        scratch_shapes=[pltpu.VMEM((1, _HEADS), jnp.float32)],
    )(P1, P1, ae)


def _xform2(P2, Wo, bo):
    def body(p0_ref, p1_ref, w_ref, b_ref, o_ref):
        p = p0_ref[...] + p1_ref[...]
        cols = []
        for h in range(_HEADS):
            s = p[:, _C1 + h]
            u = p[:, _CH * h:_CH * (h + 1)]
            cols.append(_elu(u / (s + 1e-12)[:, None]))
        hfeat = jnp.concatenate(cols, axis=1)
        zt = jnp.dot(hfeat, w_ref[...],
                     preferred_element_type=jnp.float32) + b_ref[...][None, :]
        o_ref[:, :_NCLS] = zt
        o_ref[:, _NCLS:] = jnp.zeros((_B, _D2 - _NCLS), jnp.float32)

    nb = _NP // _B
    return pl.pallas_call(
        body,
        grid=(nb,),
        in_specs=[
            pl.BlockSpec((_B, _D1), lambda j: (j, 0)),
            pl.BlockSpec((_B, _D1), lambda j, nb=nb: (nb + j, 0)),
            pl.BlockSpec((_C1, _NCLS), lambda j: (0, 0)),
            pl.BlockSpec((_NCLS,), lambda j: (0,)),
        ],
        out_specs=pl.BlockSpec((_B, _D2), lambda j: (j, 0)),
        out_shape=jax.ShapeDtypeStruct((_NP, _D2), jnp.float32),
    )(P2, P2, Wo, bo)


def _edge_combined2(P3, degc, aeo):
    nb = _MP // _B

    def body(p0_ref, p1_ref, d_ref, ae_ref, yhat_ref, m_ref):
        p_idx = pl.program_id(0)
        j = pl.program_id(1)

        @pl.when(jnp.logical_and(p_idx == 0, j == 0))
        def _():
            m_ref[...] = jnp.zeros((1, 1), jnp.float32)

        p = p0_ref[...] + p1_ref[...]
        y = p[:, :_NCLS] / d_ref[...]
        sc = _lrelu(jnp.sum(y * ae_ref[...][None, :], axis=1))[:, None]
        m_ref[...] = jnp.maximum(m_ref[...], jnp.max(sc, axis=0)[None, :])
        om_col = jnp.exp(sc - m_ref[...])
        yhat_ref[:, :_NCLS] = y * om_col
        ii = lax.broadcasted_iota(jnp.int32, (_B, _D2 - _NCLS), 1)
        yhat_ref[:, _NCLS:] = jnp.where(ii == 0, om_col, 0.0)

    return pl.pallas_call(
        body,
        grid=(2, nb),
        in_specs=[
            pl.BlockSpec((_B, _D2), lambda p, j: (j, 0)),
            pl.BlockSpec((_B, _D2), lambda p, j, nb=nb: (nb + j, 0)),
            pl.BlockSpec((_B, 1), lambda p, j: (j, 0)),
            pl.BlockSpec((_NCLS,), lambda p, j: (0,)),
        ],
        out_specs=pl.BlockSpec((_B, _D2), lambda p, j: (j, 0)),
        out_shape=jax.ShapeDtypeStruct((_MP, _D2), jnp.float32),
        scratch_shapes=[pltpu.VMEM((1, 1), jnp.float32)],
    )(P3, P3, degc, aeo)


def _finalize(P4):
    def body(p0_ref, p1_ref, o_ref):
        p = p0_ref[...] + p1_ref[...]
        s = p[:, _NCLS]
        o_ref[...] = _elu(p[:, :_NCLS] / (s + 1e-12)[:, None])

    nb = _NP // _B
    return pl.pallas_call(
        body,
        grid=(nb,),
        in_specs=[
            pl.BlockSpec((_B, _D2), lambda j: (j, 0)),
            pl.BlockSpec((_B, _D2), lambda j, nb=nb: (nb + j, 0)),
        ],
        out_specs=pl.BlockSpec((_B, _NCLS), lambda j: (j, 0)),
        out_shape=jax.ShapeDtypeStruct((_NP, _NCLS), jnp.float32),
    )(P4, P4)


def kernel(X, W, b, ae, Wo, bo, aeo, v_ids, e_ids):
    Wc = W.transpose(1, 0, 2).reshape(_C1, _C1)
    bc = b.reshape(_C1)
    v1 = v_ids.reshape(_NW, _NGRP1, _GRP1, _K1)
    e1 = e_ids.reshape(_NW, _NGRP1, _GRP1, _K1)
    v2 = v_ids.reshape(_NW, _NGRP2, _GRP2, _K2)
    e2 = e_ids.reshape(_NW, _NGRP2, _GRP2, _K2)
    X_pad = jnp.pad(X, ((0, _NP - _N), (0, 0)))

    Xt_aug = _xform1(X_pad, Wc, bc)
    P1 = _seg_sum_sc(Xt_aug, v1, e1, _MP, _D1, _K1, _GRP1, _NGRP1)
    Yhat1, degc = _edge_combined1(P1, ae)
    P2 = _seg_sum_sc(Yhat1, e1, v1, _NP, _D1, _K1, _GRP1, _NGRP1)

    Zt2_aug = _xform2(P2, Wo, bo)
    P3 = _seg_sum_sc(Zt2_aug, v2, e2, _MP, _D2, _K2, _GRP2, _NGRP2)
    Yhat2 = _edge_combined2(P3, degc, aeo)
    P4 = _seg_sum_sc(Yhat2, e2, v2, _NP, _D2, _K2, _GRP2, _NGRP2)

    out = _finalize(P4)
    return out[:_N]


# speedup vs baseline: 1.1835x; 1.0204x over previous
import functools

import jax
import jax.numpy as jnp
from jax import lax
from jax.experimental import pallas as pl
from jax.experimental.pallas import tpu as pltpu
from jax.experimental.pallas import tpu_sc as plsc

_N = 10000
_M = 5000
_P = 320000
_C1 = 128
_HEADS = 4
_CH = 32
_NCLS = 40
_NEG = 0.2

_NC = 2
_NS = 16
_NW = _NC * _NS
_K1, _GRP1, _NGRP1 = 100, 20, 5
_K2, _GRP2, _NGRP2 = 125, 80, 1

_MP = 5120
_NP = 10240
_B = 512

_D1 = 144
_D2 = 48


def _seg_sum_sc(table, gidx, sidx, n_dst_pad, D, K, grp, ngrp):
    rpt = n_dst_pad // _NS
    mesh = plsc.VectorSubcoreMesh(core_axis_name="c", subcore_axis_name="s",
                                  num_cores=_NC, num_subcores=_NS)

    def body(table_hbm, gidx_hbm, sidx_hbm, out_hbm, gv, sv,
             buf_a, buf_b, zb, acc, sem_a, sem_b, sem_z):
        c = lax.axis_index("c")
        s = lax.axis_index("s")
        wid = c * _NS + s

        def zrow(r, carry):
            for cc in range(D // 16):
                zb[r, pl.ds(cc * 16, 16)] = jnp.zeros((16,), jnp.float32)
            return carry
        lax.fori_loop(0, 16, zrow, 0)

        def zacc(k, carry):
            pltpu.async_copy(zb, acc.at[pl.ds(s * rpt + k * 16, 16)], sem_z)
            return carry
        lax.fori_loop(0, rpt // 16, zacc, 0)

        def gstart(j, buf, sem):
            pltpu.async_copy(table_hbm.at[gv.at[j]], buf, sem)

        def gwait(j, buf, sem):
            pltpu.make_async_copy(table_hbm.at[gv.at[j]], buf, sem).wait()

        pltpu.sync_copy(gidx_hbm.at[wid, 0], gv)
        pltpu.sync_copy(sidx_hbm.at[wid, 0], sv)
        gstart(0, buf_a, sem_a)

        def zwait(k, carry):
            pltpu.make_async_copy(zb, acc.at[pl.ds(s * rpt, 16)],
                                  sem_z).wait()
            return carry
        lax.fori_loop(0, rpt // 16, zwait, 0)
        plsc.subcore_barrier()

        def group(g, carry):
            @pl.when(g > 0)
            def _():
                pltpu.sync_copy(gidx_hbm.at[wid, g], gv)
                pltpu.sync_copy(sidx_hbm.at[wid, g], sv)
                gstart(0, buf_a, sem_a)

            def step(i, carry2):
                j = i * 2
                gstart(j + 1, buf_b, sem_b)
                gwait(j, buf_a, sem_a)
                pltpu.sync_copy(buf_a, acc.at[sv.at[j]], add=True)

                @pl.when(i < grp // 2 - 1)
                def _():
                    gstart(j + 2, buf_a, sem_a)

                gwait(j + 1, buf_b, sem_b)
                pltpu.sync_copy(buf_b, acc.at[sv.at[j + 1]], add=True)
                return carry2
            lax.fori_loop(0, grp // 2, step, 0)
            return carry
        lax.fori_loop(0, ngrp, group, 0)
        plsc.subcore_barrier()

        pltpu.sync_copy(acc.at[pl.ds(s * rpt, rpt)],
                        out_hbm.at[pl.ds(c * n_dst_pad + s * rpt, rpt)])

    fn = pl.kernel(
        body,
        out_type=jax.ShapeDtypeStruct((_NC * n_dst_pad, D), jnp.float32),
        mesh=mesh,
        scratch_types=[
            pltpu.VMEM((grp, K), jnp.int32),
            pltpu.VMEM((grp, K), jnp.int32),
            pltpu.VMEM((K, D), jnp.float32),
            pltpu.VMEM((K, D), jnp.float32),
            pltpu.VMEM((16, D), jnp.float32),
            pltpu.VMEM_SHARED((n_dst_pad, D), jnp.float32),
            pltpu.SemaphoreType.DMA,
            pltpu.SemaphoreType.DMA,
            pltpu.SemaphoreType.DMA,
        ],
        compiler_params=pltpu.CompilerParams(use_tc_tiling_on_sc=False),
    )
    return fn(table, gidx, sidx)


def _lrelu(x):
    return jnp.where(x >= 0, x, _NEG * x)


def _elu(x):
    return jnp.where(x > 0, x, jnp.exp(jnp.minimum(x, 0.0)) - 1.0)


def _xform1(X_pad, Wc, bc):
    def body(x_ref, w_ref, b_ref, o_ref):
        xt = jnp.dot(x_ref[...], w_ref[...],
                     preferred_element_type=jnp.float32) + b_ref[...][None, :]
        o_ref[:, :_C1] = xt
        ii = lax.broadcasted_iota(jnp.int32, (_B, _D1 - _C1), 1)
        o_ref[:, _C1:] = jnp.where(ii == 0, 1.0, 0.0)

    return pl.pallas_call(
        body,
        grid=(_NP // _B,),
        in_specs=[
            pl.BlockSpec((_B, _C1), lambda j: (j, 0)),
            pl.BlockSpec((_C1, _C1), lambda j: (0, 0)),
            pl.BlockSpec((_C1,), lambda j: (0,)),
        ],
        out_specs=pl.BlockSpec((_B, _D1), lambda j: (j, 0)),
        out_shape=jax.ShapeDtypeStruct((_NP, _D1), jnp.float32),
    )(X_pad, Wc, bc)


def _edge_combined1(P1, ae):
    nb = _MP // _B

    def body(p0_ref, p1_ref, ae_ref, yhat_ref, d_ref, m_ref):
        p_idx = pl.program_id(0)
        j = pl.program_id(1)

        @pl.when(jnp.logical_and(p_idx == 0, j == 0))
        def _():
            m_ref[...] = jnp.zeros((1, _HEADS), jnp.float32)

        p = p0_ref[...] + p1_ref[...]
        degc = jnp.clip(p[:, _C1], 1.0, None)
        d_ref[:, 0] = degc
        y = p[:, :_C1] / degc[:, None]
        sc_cols = []
        for h in range(_HEADS):
            blk = y[:, _CH * h:_CH * (h + 1)]
            a_h = jnp.sum(blk * ae_ref[h][None, :], axis=1)
            sc_cols.append(_lrelu(a_h)[:, None])
        sc = jnp.concatenate(sc_cols, axis=1)
        m_ref[...] = jnp.maximum(m_ref[...], jnp.max(sc, axis=0)[None, :])
        om_blk = jnp.exp(sc - m_ref[...])
        for h in range(_HEADS):
            yhat_ref[:, _CH * h:_CH * (h + 1)] = (
                y[:, _CH * h:_CH * (h + 1)] * om_blk[:, h][:, None])
        ii = lax.broadcasted_iota(jnp.int32, (_B, _D1 - _C1), 1)
        pad = jnp.zeros((_B, _D1 - _C1), jnp.float32)
        for h in range(_HEADS):
            pad = jnp.where(ii == h, om_blk[:, h][:, None], pad)
        yhat_ref[:, _C1:] = pad

    return pl.pallas_call(
        body,
        grid=(2, nb),
        in_specs=[
            pl.BlockSpec((_B, _D1), lambda p, j: (j, 0)),
            pl.BlockSpec((_B, _D1), lambda p, j, nb=nb: (nb + j, 0)),
            pl.BlockSpec((_HEADS, _CH), lambda p, j: (0, 0)),
        ],
        out_specs=[
            pl.BlockSpec((_B, _D1), lambda p, j: (j, 0)),
            pl.BlockSpec((_B, 1), lambda p, j: (j, 0)),
        ],
        out_shape=[
            jax.ShapeDtypeStruct((_MP, _D1), jnp.float32),
            jax.ShapeDtypeStruct((_MP, 1), jnp.float32),
        ],
        scratch_shapes=[pltpu.VMEM((1, _HEADS), jnp.float32)],
    )(P1, P1, ae)


def _xform2(P2, Wo, bo):
    def body(p0_ref, p1_ref, w_ref, b_ref, o_ref):
        p = p0_ref[...] + p1_ref[...]
        cols = []
        for h in range(_HEADS):
            s = p[:, _C1 + h]
            u = p[:, _CH * h:_CH * (h + 1)]
            cols.append(_elu(u / (s + 1e-12)[:, None]))
        hfeat = jnp.concatenate(cols, axis=1)
        zt = jnp.dot(hfeat, w_ref[...],
                     preferred_element_type=jnp.float32) + b_ref[...][None, :]
        o_ref[:, :_NCLS] = zt
        o_ref[:, _NCLS:] = jnp.zeros((_B, _D2 - _NCLS), jnp.float32)

    nb = _NP // _B
    return pl.pallas_call(
        body,
        grid=(nb,),
        in_specs=[
            pl.BlockSpec((_B, _D1), lambda j: (j, 0)),
            pl.BlockSpec((_B, _D1), lambda j, nb=nb: (nb + j, 0)),
            pl.BlockSpec((_C1, _NCLS), lambda j: (0, 0)),
            pl.BlockSpec((_NCLS,), lambda j: (0,)),
        ],
        out_specs=pl.BlockSpec((_B, _D2), lambda j: (j, 0)),
        out_shape=jax.ShapeDtypeStruct((_NP, _D2), jnp.float32),
    )(P2, P2, Wo, bo)


def _edge_combined2(P3, degc, aeo):
    nb = _MP // _B

    def body(p0_ref, p1_ref, d_ref, ae_ref, yhat_ref, m_ref):
        p_idx = pl.program_id(0)
        j = pl.program_id(1)

        @pl.when(jnp.logical_and(p_idx == 0, j == 0))
        def _():
            m_ref[...] = jnp.zeros((1, 1), jnp.float32)

        p = p0_ref[...] + p1_ref[...]
        y = p[:, :_NCLS] / d_ref[...]
        sc = _lrelu(jnp.sum(y * ae_ref[...][None, :], axis=1))[:, None]
        m_ref[...] = jnp.maximum(m_ref[...], jnp.max(sc, axis=0)[None, :])
        om_col = jnp.exp(sc - m_ref[...])
        yhat_ref[:, :_NCLS] = y * om_col
        ii = lax.broadcasted_iota(jnp.int32, (_B, _D2 - _NCLS), 1)
        yhat_ref[:, _NCLS:] = jnp.where(ii == 0, om_col, 0.0)

    return pl.pallas_call(
        body,
        grid=(2, nb),
        in_specs=[
            pl.BlockSpec((_B, _D2), lambda p, j: (j, 0)),
            pl.BlockSpec((_B, _D2), lambda p, j, nb=nb: (nb + j, 0)),
            pl.BlockSpec((_B, 1), lambda p, j: (j, 0)),
            pl.BlockSpec((_NCLS,), lambda p, j: (0,)),
        ],
        out_specs=pl.BlockSpec((_B, _D2), lambda p, j: (j, 0)),
        out_shape=jax.ShapeDtypeStruct((_MP, _D2), jnp.float32),
        scratch_shapes=[pltpu.VMEM((1, 1), jnp.float32)],
    )(P3, P3, degc, aeo)


def _finalize(P4):
    def body(p0_ref, p1_ref, o_ref):
        p = p0_ref[...] + p1_ref[...]
        s = p[:, _NCLS]
        o_ref[...] = _elu(p[:, :_NCLS] / (s + 1e-12)[:, None])

    nb = _NP // _B
    return pl.pallas_call(
        body,
        grid=(nb,),
        in_specs=[
            pl.BlockSpec((_B, _D2), lambda j: (j, 0)),
            pl.BlockSpec((_B, _D2), lambda j, nb=nb: (nb + j, 0)),
        ],
        out_specs=pl.BlockSpec((_B, _NCLS), lambda j: (j, 0)),
        out_shape=jax.ShapeDtypeStruct((_NP, _NCLS), jnp.float32),
    )(P4, P4)


def kernel(X, W, b, ae, Wo, bo, aeo, v_ids, e_ids):
    Wc = W.transpose(1, 0, 2).reshape(_C1, _C1)
    bc = b.reshape(_C1)
    v1 = v_ids.reshape(_NW, _NGRP1, _GRP1, _K1)
    e1 = e_ids.reshape(_NW, _NGRP1, _GRP1, _K1)
    v2 = v_ids.reshape(_NW, _NGRP2, _GRP2, _K2)
    e2 = e_ids.reshape(_NW, _NGRP2, _GRP2, _K2)
    X_pad = jnp.pad(X, ((0, _NP - _N), (0, 0)))

    Xt_aug = _xform1(X_pad, Wc, bc)
    P1 = _seg_sum_sc(Xt_aug, v1, e1, _MP, _D1, _K1, _GRP1, _NGRP1)
    Yhat1, degc = _edge_combined1(P1, ae)
    P2 = _seg_sum_sc(Yhat1, e1, v1, _NP, _D1, _K1, _GRP1, _NGRP1)

    Zt2_aug = _xform2(P2, Wo, bo)
    P3 = _seg_sum_sc(Zt2_aug, v2, e2, _MP, _D2, _K2, _GRP2, _NGRP2)
    Yhat2 = _edge_combined2(P3, degc, aeo)
    P4 = _seg_sum_sc(Yhat2, e2, v2, _NP, _D2, _K2, _GRP2, _NGRP2)

    out = _finalize(P4)
    return out[:_N]


# speedup vs baseline: 1.1960x; 1.0106x over previous
import functools

import jax
import jax.numpy as jnp
from jax import lax
from jax.experimental import pallas as pl
from jax.experimental.pallas import tpu as pltpu
from jax.experimental.pallas import tpu_sc as plsc

_N = 10000
_M = 5000
_P = 320000
_C1 = 128
_HEADS = 4
_CH = 32
_NCLS = 40
_NEG = 0.2

_NC = 2
_NS = 16
_NW = _NC * _NS
_K1, _GRP1, _NGRP1 = 100, 20, 5
_K2, _GRP2, _NGRP2 = 125, 80, 1

_MP = 5120
_NP = 10240
_B = 512

_D1 = 144
_D2 = 48


def _seg_sum_sc(table, gidx, sidx, n_dst_pad, D, K, grp, ngrp):
    rpt = n_dst_pad // _NS
    mesh = plsc.VectorSubcoreMesh(core_axis_name="c", subcore_axis_name="s",
                                  num_cores=_NC, num_subcores=_NS)

    def body(table_hbm, gidx_hbm, sidx_hbm, out_hbm, gv, sv,
             buf_a, buf_b, zb, acc, sem_a, sem_b, sem_z):
        c = lax.axis_index("c")
        s = lax.axis_index("s")
        wid = c * _NS + s

        def zrow(r, carry):
            for cc in range(D // 16):
                zb[r, pl.ds(cc * 16, 16)] = jnp.zeros((16,), jnp.float32)
            return carry
        lax.fori_loop(0, 16, zrow, 0)

        def zacc(k, carry):
            pltpu.async_copy(zb, acc.at[pl.ds(s * rpt + k * 16, 16)], sem_z)
            return carry
        lax.fori_loop(0, rpt // 16, zacc, 0)

        def gstart(j, buf, sem):
            pltpu.async_copy(table_hbm.at[gv.at[j]], buf, sem)

        def gwait(j, buf, sem):
            pltpu.make_async_copy(table_hbm.at[gv.at[j]], buf, sem).wait()

        pltpu.sync_copy(gidx_hbm.at[wid, 0], gv)
        pltpu.sync_copy(sidx_hbm.at[wid, 0], sv)
        gstart(0, buf_a, sem_a)

        def zwait(k, carry):
            pltpu.make_async_copy(zb, acc.at[pl.ds(s * rpt, 16)],
                                  sem_z).wait()
            return carry
        lax.fori_loop(0, rpt // 16, zwait, 0)
        plsc.subcore_barrier()

        def group(g, carry):
            @pl.when(g > 0)
            def _():
                pltpu.sync_copy(gidx_hbm.at[wid, g], gv)
                pltpu.sync_copy(sidx_hbm.at[wid, g], sv)
                gstart(0, buf_a, sem_a)

            def step(i, carry2):
                j = i * 2
                gstart(j + 1, buf_b, sem_b)
                gwait(j, buf_a, sem_a)
                pltpu.sync_copy(buf_a, acc.at[sv.at[j]], add=True)

                @pl.when(i < grp // 2 - 1)
                def _():
                    gstart(j + 2, buf_a, sem_a)

                gwait(j + 1, buf_b, sem_b)
                pltpu.sync_copy(buf_b, acc.at[sv.at[j + 1]], add=True)
                return carry2
            lax.fori_loop(0, grp // 2, step, 0)
            return carry
        lax.fori_loop(0, ngrp, group, 0)
        plsc.subcore_barrier()

        pltpu.sync_copy(acc.at[pl.ds(s * rpt, rpt)],
                        out_hbm.at[pl.ds(c * n_dst_pad + s * rpt, rpt)])

    fn = pl.kernel(
        body,
        out_type=jax.ShapeDtypeStruct((_NC * n_dst_pad, D), jnp.float32),
        mesh=mesh,
        scratch_types=[
            pltpu.VMEM((grp, K), jnp.int32),
            pltpu.VMEM((grp, K), jnp.int32),
            pltpu.VMEM((K, D), jnp.float32),
            pltpu.VMEM((K, D), jnp.float32),
            pltpu.VMEM((16, D), jnp.float32),
            pltpu.VMEM_SHARED((n_dst_pad, D), jnp.float32),
            pltpu.SemaphoreType.DMA,
            pltpu.SemaphoreType.DMA,
            pltpu.SemaphoreType.DMA,
        ],
        compiler_params=pltpu.CompilerParams(use_tc_tiling_on_sc=False),
    )
    return fn(table, gidx, sidx)


def _lrelu(x):
    return jnp.where(x >= 0, x, _NEG * x)


def _elu(x):
    return jnp.where(x > 0, x, jnp.exp(jnp.minimum(x, 0.0)) - 1.0)


def _xform1(X_pad, Wc, bc):
    def body(x_ref, w_ref, b_ref, o_ref):
        xt = jnp.dot(x_ref[...], w_ref[...],
                     preferred_element_type=jnp.float32) + b_ref[...][None, :]
        o_ref[:, :_C1] = xt
        ii = lax.broadcasted_iota(jnp.int32, (_B, _D1 - _C1), 1)
        o_ref[:, _C1:] = jnp.where(ii == 0, 1.0, 0.0)

    return pl.pallas_call(
        body,
        grid=(_NP // _B,),
        in_specs=[
            pl.BlockSpec((_B, _C1), lambda j: (j, 0)),
            pl.BlockSpec((_C1, _C1), lambda j: (0, 0)),
            pl.BlockSpec((_C1,), lambda j: (0,)),
        ],
        out_specs=pl.BlockSpec((_B, _D1), lambda j: (j, 0)),
        out_shape=jax.ShapeDtypeStruct((_NP, _D1), jnp.float32),
    )(X_pad, Wc, bc)


def _edge_combined1(P1, ae):
    nb = _MP // _B

    def body(p0_ref, p1_ref, ae_ref, yhat_ref, d_ref, m_ref):
        p_idx = pl.program_id(0)
        j = pl.program_id(1)

        @pl.when(jnp.logical_and(p_idx == 0, j == 0))
        def _():
            m_ref[...] = jnp.zeros((1, _HEADS), jnp.float32)

        p = p0_ref[...] + p1_ref[...]
        degc = jnp.clip(p[:, _C1], 1.0, None)
        d_ref[:, 0] = degc
        y = p[:, :_C1] / degc[:, None]
        sc = _lrelu(jnp.dot(y, ae_ref[...],
                            preferred_element_type=jnp.float32))
        m_ref[...] = jnp.maximum(m_ref[...], jnp.max(sc, axis=0)[None, :])
        om_blk = jnp.exp(sc - m_ref[...])
        for h in range(_HEADS):
            yhat_ref[:, _CH * h:_CH * (h + 1)] = (
                y[:, _CH * h:_CH * (h + 1)] * om_blk[:, h][:, None])
        ii = lax.broadcasted_iota(jnp.int32, (_B, _D1 - _C1), 1)
        pad = jnp.zeros((_B, _D1 - _C1), jnp.float32)
        for h in range(_HEADS):
            pad = jnp.where(ii == h, om_blk[:, h][:, None], pad)
        yhat_ref[:, _C1:] = pad

    return pl.pallas_call(
        body,
        grid=(2, nb),
        in_specs=[
            pl.BlockSpec((_B, _D1), lambda p, j: (j, 0)),
            pl.BlockSpec((_B, _D1), lambda p, j, nb=nb: (nb + j, 0)),
            pl.BlockSpec((_C1, _HEADS), lambda p, j: (0, 0)),
        ],
        out_specs=[
            pl.BlockSpec((_B, _D1), lambda p, j: (j, 0)),
            pl.BlockSpec((_B, 1), lambda p, j: (j, 0)),
        ],
        out_shape=[
            jax.ShapeDtypeStruct((_MP, _D1), jnp.float32),
            jax.ShapeDtypeStruct((_MP, 1), jnp.float32),
        ],
        scratch_shapes=[pltpu.VMEM((1, _HEADS), jnp.float32)],
    )(P1, P1, ae)


def _xform2(P2, Wo, bo):
    def body(p0_ref, p1_ref, w_ref, b_ref, o_ref):
        p = p0_ref[...] + p1_ref[...]
        cols = []
        for h in range(_HEADS):
            s = p[:, _C1 + h]
            u = p[:, _CH * h:_CH * (h + 1)]
            cols.append(_elu(u / (s + 1e-12)[:, None]))
        hfeat = jnp.concatenate(cols, axis=1)
        zt = jnp.dot(hfeat, w_ref[...],
                     preferred_element_type=jnp.float32) + b_ref[...][None, :]
        o_ref[:, :_NCLS] = zt
        o_ref[:, _NCLS:] = jnp.zeros((_B, _D2 - _NCLS), jnp.float32)

    nb = _NP // _B
    return pl.pallas_call(
        body,
        grid=(nb,),
        in_specs=[
            pl.BlockSpec((_B, _D1), lambda j: (j, 0)),
            pl.BlockSpec((_B, _D1), lambda j, nb=nb: (nb + j, 0)),
            pl.BlockSpec((_C1, _NCLS), lambda j: (0, 0)),
            pl.BlockSpec((_NCLS,), lambda j: (0,)),
        ],
        out_specs=pl.BlockSpec((_B, _D2), lambda j: (j, 0)),
        out_shape=jax.ShapeDtypeStruct((_NP, _D2), jnp.float32),
    )(P2, P2, Wo, bo)


def _edge_combined2(P3, degc, aeo):
    nb = _MP // _B

    def body(p0_ref, p1_ref, d_ref, ae_ref, yhat_ref, m_ref):
        p_idx = pl.program_id(0)
        j = pl.program_id(1)

        @pl.when(jnp.logical_and(p_idx == 0, j == 0))
        def _():
            m_ref[...] = jnp.zeros((1, 1), jnp.float32)

        p = p0_ref[...] + p1_ref[...]
        y = p[:, :_NCLS] / d_ref[...]
        sc = _lrelu(jnp.dot(y, ae_ref[...],
                            preferred_element_type=jnp.float32))
        m_ref[...] = jnp.maximum(m_ref[...], jnp.max(sc, axis=0)[None, :])
        om_col = jnp.exp(sc - m_ref[...])
        yhat_ref[:, :_NCLS] = y * om_col
        ii = lax.broadcasted_iota(jnp.int32, (_B, _D2 - _NCLS), 1)
        yhat_ref[:, _NCLS:] = jnp.where(ii == 0, om_col, 0.0)

    return pl.pallas_call(
        body,
        grid=(2, nb),
        in_specs=[
            pl.BlockSpec((_B, _D2), lambda p, j: (j, 0)),
            pl.BlockSpec((_B, _D2), lambda p, j, nb=nb: (nb + j, 0)),
            pl.BlockSpec((_B, 1), lambda p, j: (j, 0)),
            pl.BlockSpec((_NCLS, 1), lambda p, j: (0, 0)),
        ],
        out_specs=pl.BlockSpec((_B, _D2), lambda p, j: (j, 0)),
        out_shape=jax.ShapeDtypeStruct((_MP, _D2), jnp.float32),
        scratch_shapes=[pltpu.VMEM((1, 1), jnp.float32)],
    )(P3, P3, degc, aeo)


def _finalize(P4):
    def body(p0_ref, p1_ref, o_ref):
        p = p0_ref[...] + p1_ref[...]
        s = p[:, _NCLS]
        o_ref[...] = _elu(p[:, :_NCLS] / (s + 1e-12)[:, None])

    nb = _NP // _B
    return pl.pallas_call(
        body,
        grid=(nb,),
        in_specs=[
            pl.BlockSpec((_B, _D2), lambda j: (j, 0)),
            pl.BlockSpec((_B, _D2), lambda j, nb=nb: (nb + j, 0)),
        ],
        out_specs=pl.BlockSpec((_B, _NCLS), lambda j: (j, 0)),
        out_shape=jax.ShapeDtypeStruct((_NP, _NCLS), jnp.float32),
    )(P4, P4)


def kernel(X, W, b, ae, Wo, bo, aeo, v_ids, e_ids):
    Wc = W.transpose(1, 0, 2).reshape(_C1, _C1)
    bc = b.reshape(_C1)
    ae_mat = jnp.zeros((_C1, _HEADS), jnp.float32)
    for h in range(_HEADS):
        ae_mat = ae_mat.at[_CH * h:_CH * (h + 1), h].set(ae[h])
    aeo_mat = aeo[:, None]
    v1 = v_ids.reshape(_NW, _NGRP1, _GRP1, _K1)
    e1 = e_ids.reshape(_NW, _NGRP1, _GRP1, _K1)
    v2 = v_ids.reshape(_NW, _NGRP2, _GRP2, _K2)
    e2 = e_ids.reshape(_NW, _NGRP2, _GRP2, _K2)
    X_pad = jnp.pad(X, ((0, _NP - _N), (0, 0)))

    Xt_aug = _xform1(X_pad, Wc, bc)
    P1 = _seg_sum_sc(Xt_aug, v1, e1, _MP, _D1, _K1, _GRP1, _NGRP1)
    Yhat1, degc = _edge_combined1(P1, ae_mat)
    P2 = _seg_sum_sc(Yhat1, e1, v1, _NP, _D1, _K1, _GRP1, _NGRP1)

    Zt2_aug = _xform2(P2, Wo, bo)
    P3 = _seg_sum_sc(Zt2_aug, v2, e2, _MP, _D2, _K2, _GRP2, _NGRP2)
    Yhat2 = _edge_combined2(P3, degc, aeo_mat)
    P4 = _seg_sum_sc(Yhat2, e2, v2, _NP, _D2, _K2, _GRP2, _NGRP2)

    out = _finalize(P4)
    return out[:_N]


# speedup vs baseline: 1.2755x; 1.0665x over previous
import functools

import jax
import jax.numpy as jnp
from jax import lax
from jax.experimental import pallas as pl
from jax.experimental.pallas import tpu as pltpu
from jax.experimental.pallas import tpu_sc as plsc

_N = 10000
_M = 5000
_P = 320000
_C1 = 128
_HEADS = 4
_CH = 32
_NCLS = 40
_NEG = 0.2

_NC = 2
_NS = 16
_NW = _NC * _NS
_K1, _GRP1, _NGRP1 = 100, 20, 5
_K2, _GRP2, _NGRP2 = 125, 80, 1

_MP = 5120
_NP = 10240
_B = 512

_D1 = 144
_D2 = 48


def _seg_sum_sc(table, gidx, sidx, n_dst_pad, D, K, grp, ngrp, nbuf=2):
    rpt = n_dst_pad // _NS
    mesh = plsc.VectorSubcoreMesh(core_axis_name="c", subcore_axis_name="s",
                                  num_cores=_NC, num_subcores=_NS)

    def body(table_hbm, gidx_hbm, sidx_hbm, out_hbm, *scr):
        gv, sv = scr[0], scr[1]
        bufs = scr[2:2 + nbuf]
        zb, acc = scr[2 + nbuf], scr[3 + nbuf]
        sems = scr[4 + nbuf:4 + 2 * nbuf]
        sem_z = scr[4 + 2 * nbuf]
        c = lax.axis_index("c")
        s = lax.axis_index("s")
        wid = c * _NS + s

        def zrow(r, carry):
            for cc in range(D // 16):
                zb[r, pl.ds(cc * 16, 16)] = jnp.zeros((16,), jnp.float32)
            return carry
        lax.fori_loop(0, 16, zrow, 0)

        def zacc(k, carry):
            pltpu.async_copy(zb, acc.at[pl.ds(s * rpt + k * 16, 16)], sem_z)
            return carry
        lax.fori_loop(0, rpt // 16, zacc, 0)

        def gstart(j, buf, sem):
            pltpu.async_copy(table_hbm.at[gv.at[j]], buf, sem)

        def gwait(j, buf, sem):
            pltpu.make_async_copy(table_hbm.at[gv.at[j]], buf, sem).wait()

        pltpu.sync_copy(gidx_hbm.at[wid, 0], gv)
        pltpu.sync_copy(sidx_hbm.at[wid, 0], sv)
        for b in range(nbuf - 1):
            gstart(b, bufs[b], sems[b])

        def zwait(k, carry):
            pltpu.make_async_copy(zb, acc.at[pl.ds(s * rpt, 16)],
                                  sem_z).wait()
            return carry
        lax.fori_loop(0, rpt // 16, zwait, 0)
        plsc.subcore_barrier()

        def group(g, carry):
            @pl.when(g > 0)
            def _():
                pltpu.sync_copy(gidx_hbm.at[wid, g], gv)
                pltpu.sync_copy(sidx_hbm.at[wid, g], sv)
                for b in range(nbuf - 1):
                    gstart(b, bufs[b], sems[b])

            def step(i, carry2):
                for b in range(nbuf):
                    j = i * nbuf + b
                    gstart(j + nbuf - 1, bufs[(b + nbuf - 1) % nbuf],
                           sems[(b + nbuf - 1) % nbuf])
                    gwait(j, bufs[b], sems[b])
                    pltpu.sync_copy(bufs[b], acc.at[sv.at[j]], add=True)
                return carry2
            lax.fori_loop(0, grp // nbuf - 1, step, 0)

            for b in range(nbuf):
                j = grp - nbuf + b
                if b == 0:
                    gstart(grp - 1, bufs[nbuf - 1], sems[nbuf - 1])
                gwait(j, bufs[b], sems[b])
                pltpu.sync_copy(bufs[b], acc.at[sv.at[j]], add=True)
            return carry
        lax.fori_loop(0, ngrp, group, 0)
        plsc.subcore_barrier()

        pltpu.sync_copy(acc.at[pl.ds(s * rpt, rpt)],
                        out_hbm.at[pl.ds(c * n_dst_pad + s * rpt, rpt)])

    fn = pl.kernel(
        body,
        out_type=jax.ShapeDtypeStruct((_NC * n_dst_pad, D), jnp.float32),
        mesh=mesh,
        scratch_types=(
            [pltpu.VMEM((grp, K), jnp.int32),
             pltpu.VMEM((grp, K), jnp.int32)]
            + [pltpu.VMEM((K, D), jnp.float32)] * nbuf
            + [pltpu.VMEM((16, D), jnp.float32),
               pltpu.VMEM_SHARED((n_dst_pad, D), jnp.float32)]
            + [pltpu.SemaphoreType.DMA] * (nbuf + 1)
        ),
        compiler_params=pltpu.CompilerParams(use_tc_tiling_on_sc=False),
    )
    return fn(table, gidx, sidx)


def _lrelu(x):
    return jnp.where(x >= 0, x, _NEG * x)


def _elu(x):
    return jnp.where(x > 0, x, jnp.exp(jnp.minimum(x, 0.0)) - 1.0)


def _xform1(X_pad, Wc, bc):
    def body(x_ref, w_ref, b_ref, o_ref):
        xt = jnp.dot(x_ref[...], w_ref[...],
                     preferred_element_type=jnp.float32) + b_ref[...][None, :]
        o_ref[:, :_C1] = xt
        ii = lax.broadcasted_iota(jnp.int32, (_B, _D1 - _C1), 1)
        o_ref[:, _C1:] = jnp.where(ii == 0, 1.0, 0.0)

    return pl.pallas_call(
        body,
        grid=(_NP // _B,),
        in_specs=[
            pl.BlockSpec((_B, _C1), lambda j: (j, 0)),
            pl.BlockSpec((_C1, _C1), lambda j: (0, 0)),
            pl.BlockSpec((_C1,), lambda j: (0,)),
        ],
        out_specs=pl.BlockSpec((_B, _D1), lambda j: (j, 0)),
        out_shape=jax.ShapeDtypeStruct((_NP, _D1), jnp.float32),
    )(X_pad, Wc, bc)


def _edge_combined1(P1, ae):
    nb = _MP // _B

    def body(p0_ref, p1_ref, ae_ref, yhat_ref, d_ref, m_ref):
        p_idx = pl.program_id(0)
        j = pl.program_id(1)

        @pl.when(jnp.logical_and(p_idx == 0, j == 0))
        def _():
            m_ref[...] = jnp.zeros((1, _HEADS), jnp.float32)

        p = p0_ref[...] + p1_ref[...]
        degc = jnp.clip(p[:, _C1], 1.0, None)
        d_ref[:, 0] = degc
        y = p[:, :_C1] / degc[:, None]
        sc = _lrelu(jnp.dot(y, ae_ref[...],
                            preferred_element_type=jnp.float32))
        m_ref[...] = jnp.maximum(m_ref[...], jnp.max(sc, axis=0)[None, :])
        om_blk = jnp.exp(sc - m_ref[...])
        for h in range(_HEADS):
            yhat_ref[:, _CH * h:_CH * (h + 1)] = (
                y[:, _CH * h:_CH * (h + 1)] * om_blk[:, h][:, None])
        ii = lax.broadcasted_iota(jnp.int32, (_B, _D1 - _C1), 1)
        pad = jnp.zeros((_B, _D1 - _C1), jnp.float32)
        for h in range(_HEADS):
            pad = jnp.where(ii == h, om_blk[:, h][:, None], pad)
        yhat_ref[:, _C1:] = pad

    return pl.pallas_call(
        body,
        grid=(2, nb),
        in_specs=[
            pl.BlockSpec((_B, _D1), lambda p, j: (j, 0)),
            pl.BlockSpec((_B, _D1), lambda p, j, nb=nb: (nb + j, 0)),
            pl.BlockSpec((_C1, _HEADS), lambda p, j: (0, 0)),
        ],
        out_specs=[
            pl.BlockSpec((_B, _D1), lambda p, j: (j, 0)),
            pl.BlockSpec((_B, 1), lambda p, j: (j, 0)),
        ],
        out_shape=[
            jax.ShapeDtypeStruct((_MP, _D1), jnp.float32),
            jax.ShapeDtypeStruct((_MP, 1), jnp.float32),
        ],
        scratch_shapes=[pltpu.VMEM((1, _HEADS), jnp.float32)],
    )(P1, P1, ae)


def _xform2(P2, Wo, bo):
    def body(p0_ref, p1_ref, w_ref, b_ref, o_ref):
        p = p0_ref[...] + p1_ref[...]
        cols = []
        for h in range(_HEADS):
            s = p[:, _C1 + h]
            u = p[:, _CH * h:_CH * (h + 1)]
            cols.append(_elu(u / (s + 1e-12)[:, None]))
        hfeat = jnp.concatenate(cols, axis=1)
        zt = jnp.dot(hfeat, w_ref[...],
                     preferred_element_type=jnp.float32) + b_ref[...][None, :]
        o_ref[:, :_NCLS] = zt
        o_ref[:, _NCLS:] = jnp.zeros((_B, _D2 - _NCLS), jnp.float32)

    nb = _NP // _B
    return pl.pallas_call(
        body,
        grid=(nb,),
        in_specs=[
            pl.BlockSpec((_B, _D1), lambda j: (j, 0)),
            pl.BlockSpec((_B, _D1), lambda j, nb=nb: (nb + j, 0)),
            pl.BlockSpec((_C1, _NCLS), lambda j: (0, 0)),
            pl.BlockSpec((_NCLS,), lambda j: (0,)),
        ],
        out_specs=pl.BlockSpec((_B, _D2), lambda j: (j, 0)),
        out_shape=jax.ShapeDtypeStruct((_NP, _D2), jnp.float32),
    )(P2, P2, Wo, bo)


def _edge_combined2(P3, degc, aeo):
    nb = _MP // _B

    def body(p0_ref, p1_ref, d_ref, ae_ref, yhat_ref, m_ref):
        p_idx = pl.program_id(0)
        j = pl.program_id(1)

        @pl.when(jnp.logical_and(p_idx == 0, j == 0))
        def _():
            m_ref[...] = jnp.zeros((1, 1), jnp.float32)

        p = p0_ref[...] + p1_ref[...]
        y = p[:, :_NCLS] / d_ref[...]
        sc = _lrelu(jnp.dot(y, ae_ref[...],
                            preferred_element_type=jnp.float32))
        m_ref[...] = jnp.maximum(m_ref[...], jnp.max(sc, axis=0)[None, :])
        om_col = jnp.exp(sc - m_ref[...])
        yhat_ref[:, :_NCLS] = y * om_col
        ii = lax.broadcasted_iota(jnp.int32, (_B, _D2 - _NCLS), 1)
        yhat_ref[:, _NCLS:] = jnp.where(ii == 0, om_col, 0.0)

    return pl.pallas_call(
        body,
        grid=(2, nb),
        in_specs=[
            pl.BlockSpec((_B, _D2), lambda p, j: (j, 0)),
            pl.BlockSpec((_B, _D2), lambda p, j, nb=nb: (nb + j, 0)),
            pl.BlockSpec((_B, 1), lambda p, j: (j, 0)),
            pl.BlockSpec((_NCLS, 1), lambda p, j: (0, 0)),
        ],
        out_specs=pl.BlockSpec((_B, _D2), lambda p, j: (j, 0)),
        out_shape=jax.ShapeDtypeStruct((_MP, _D2), jnp.float32),
        scratch_shapes=[pltpu.VMEM((1, 1), jnp.float32)],
    )(P3, P3, degc, aeo)


def _finalize(P4):
    def body(p0_ref, p1_ref, o_ref):
        p = p0_ref[...] + p1_ref[...]
        s = p[:, _NCLS]
        o_ref[...] = _elu(p[:, :_NCLS] / (s + 1e-12)[:, None])

    nb = _NP // _B
    return pl.pallas_call(
        body,
        grid=(nb,),
        in_specs=[
            pl.BlockSpec((_B, _D2), lambda j: (j, 0)),
            pl.BlockSpec((_B, _D2), lambda j, nb=nb: (nb + j, 0)),
        ],
        out_specs=pl.BlockSpec((_B, _NCLS), lambda j: (j, 0)),
        out_shape=jax.ShapeDtypeStruct((_NP, _NCLS), jnp.float32),
    )(P4, P4)


def kernel(X, W, b, ae, Wo, bo, aeo, v_ids, e_ids):
    Wc = W.transpose(1, 0, 2).reshape(_C1, _C1)
    bc = b.reshape(_C1)
    ae_mat = jnp.zeros((_C1, _HEADS), jnp.float32)
    for h in range(_HEADS):
        ae_mat = ae_mat.at[_CH * h:_CH * (h + 1), h].set(ae[h])
    aeo_mat = aeo[:, None]
    v1 = v_ids.reshape(_NW, _NGRP1, _GRP1, _K1)
    e1 = e_ids.reshape(_NW, _NGRP1, _GRP1, _K1)
    v2 = v_ids.reshape(_NW, _NGRP2, _GRP2, _K2)
    e2 = e_ids.reshape(_NW, _NGRP2, _GRP2, _K2)
    X_pad = jnp.pad(X, ((0, _NP - _N), (0, 0)))

    Xt_aug = _xform1(X_pad, Wc, bc)
    P1 = _seg_sum_sc(Xt_aug, v1, e1, _MP, _D1, _K1, _GRP1, _NGRP1)
    Yhat1, degc = _edge_combined1(P1, ae_mat)
    P2 = _seg_sum_sc(Yhat1, e1, v1, _NP, _D1, _K1, _GRP1, _NGRP1)

    Zt2_aug = _xform2(P2, Wo, bo)
    P3 = _seg_sum_sc(Zt2_aug, v2, e2, _MP, _D2, _K2, _GRP2, _NGRP2, nbuf=4)
    Yhat2 = _edge_combined2(P3, degc, aeo_mat)
    P4 = _seg_sum_sc(Yhat2, e2, v2, _NP, _D2, _K2, _GRP2, _NGRP2, nbuf=4)

    out = _finalize(P4)
    return out[:_N]


# speedup vs baseline: 1.2781x; 1.0020x over previous
import jax
import jax.numpy as jnp
from jax import lax
from jax.experimental import pallas as pl
from jax.experimental.pallas import tpu as pltpu
from jax.experimental.pallas import tpu_sc as plsc

_N = 10000
_M = 5000
_P = 320000
_C1 = 128
_HEADS = 4
_CH = 32
_NCLS = 40
_NEG = 0.2

_NC = 2
_NS = 16
_NW = _NC * _NS
_K1, _GRP1, _NGRP1 = 100, 20, 5
_K2, _GRP2, _NGRP2 = 125, 80, 1

_MP = 5120
_NP = 10240
_B = 512

_D1 = 144
_D2 = 48


def _seg_sum_sc(table, gidx, sidx, n_dst_pad, D, K, grp, ngrp, nbuf=2):
    rpt = n_dst_pad // _NS
    mesh = plsc.VectorSubcoreMesh(core_axis_name="c", subcore_axis_name="s",
                                  num_cores=_NC, num_subcores=_NS)

    def body(table_hbm, gidx_hbm, sidx_hbm, out_hbm, *scr):
        gv, sv = scr[0], scr[1]
        bufs = scr[2:2 + nbuf]
        zb, acc = scr[2 + nbuf], scr[3 + nbuf]
        sems = scr[4 + nbuf:4 + 2 * nbuf]
        sem_z = scr[4 + 2 * nbuf]
        c = lax.axis_index("c")
        s = lax.axis_index("s")
        wid = c * _NS + s

        def zrow(r, carry):
            for cc in range(D // 16):
                zb[r, pl.ds(cc * 16, 16)] = jnp.zeros((16,), jnp.float32)
            return carry
        lax.fori_loop(0, 16, zrow, 0)

        def zacc(k, carry):
            pltpu.async_copy(zb, acc.at[pl.ds(s * rpt + k * 16, 16)], sem_z)
            return carry
        lax.fori_loop(0, rpt // 16, zacc, 0)

        def gstart(j, buf, sem):
            pltpu.async_copy(table_hbm.at[gv.at[j]], buf, sem)

        def gwait(j, buf, sem):
            pltpu.make_async_copy(table_hbm.at[gv.at[j]], buf, sem).wait()

        pltpu.sync_copy(gidx_hbm.at[wid, 0], gv)
        pltpu.sync_copy(sidx_hbm.at[wid, 0], sv)
        for b in range(nbuf - 1):
            gstart(b, bufs[b], sems[b])

        def zwait(k, carry):
            pltpu.make_async_copy(zb, acc.at[pl.ds(s * rpt, 16)],
                                  sem_z).wait()
            return carry
        lax.fori_loop(0, rpt // 16, zwait, 0)
        plsc.subcore_barrier()

        def group(g, carry):
            @pl.when(g > 0)
            def _():
                pltpu.sync_copy(gidx_hbm.at[wid, g], gv)
                pltpu.sync_copy(sidx_hbm.at[wid, g], sv)
                for b in range(nbuf - 1):
                    gstart(b, bufs[b], sems[b])

            def step(i, carry2):
                for b in range(nbuf):
                    j = i * nbuf + b
                    gstart(j + nbuf - 1, bufs[(b + nbuf - 1) % nbuf],
                           sems[(b + nbuf - 1) % nbuf])
                    gwait(j, bufs[b], sems[b])
                    pltpu.sync_copy(bufs[b], acc.at[sv.at[j]], add=True)
                return carry2
            lax.fori_loop(0, grp // nbuf - 1, step, 0)

            for b in range(nbuf):
                j = grp - nbuf + b
                if b == 0:
                    gstart(grp - 1, bufs[nbuf - 1], sems[nbuf - 1])
                gwait(j, bufs[b], sems[b])
                pltpu.sync_copy(bufs[b], acc.at[sv.at[j]], add=True)
            return carry
        lax.fori_loop(0, ngrp, group, 0)
        plsc.subcore_barrier()

        pltpu.sync_copy(acc.at[pl.ds(s * rpt, rpt)],
                        out_hbm.at[pl.ds(c * n_dst_pad + s * rpt, rpt)])

    fn = pl.kernel(
        body,
        out_type=jax.ShapeDtypeStruct((_NC * n_dst_pad, D), jnp.float32),
        mesh=mesh,
        scratch_types=(
            [pltpu.VMEM((grp, K), jnp.int32),
             pltpu.VMEM((grp, K), jnp.int32)]
            + [pltpu.VMEM((K, D), jnp.float32)] * nbuf
            + [pltpu.VMEM((16, D), jnp.float32),
               pltpu.VMEM_SHARED((n_dst_pad, D), jnp.float32)]
            + [pltpu.SemaphoreType.DMA] * (nbuf + 1)
        ),
        compiler_params=pltpu.CompilerParams(use_tc_tiling_on_sc=False),
    )
    return fn(table, gidx, sidx)


def _lrelu(x):
    return jnp.where(x >= 0, x, _NEG * x)


def _elu(x):
    return jnp.where(x > 0, x, jnp.exp(jnp.minimum(x, 0.0)) - 1.0)


def _xform1(X_pad, Wc, bc):
    def body(x_ref, w_ref, b_ref, o_ref):
        xt = jnp.dot(x_ref[...], w_ref[...],
                     preferred_element_type=jnp.float32) + b_ref[...][None, :]
        o_ref[:, :_C1] = xt
        ii = lax.broadcasted_iota(jnp.int32, (_B, _D1 - _C1), 1)
        o_ref[:, _C1:] = jnp.where(ii == 0, 1.0, 0.0)

    return pl.pallas_call(
        body,
        grid=(_NP // _B,),
        in_specs=[
            pl.BlockSpec((_B, _C1), lambda j: (j, 0)),
            pl.BlockSpec((_C1, _C1), lambda j: (0, 0)),
            pl.BlockSpec((_C1,), lambda j: (0,)),
        ],
        out_specs=pl.BlockSpec((_B, _D1), lambda j: (j, 0)),
        out_shape=jax.ShapeDtypeStruct((_NP, _D1), jnp.float32),
    )(X_pad, Wc, bc)


def _edge_combined1(P1, ae):
    nb = _MP // _B

    def body(p0_ref, p1_ref, ae_ref, yhat_ref, d_ref, m_ref):
        p_idx = pl.program_id(0)
        j = pl.program_id(1)

        @pl.when(jnp.logical_and(p_idx == 0, j == 0))
        def _():
            m_ref[...] = jnp.zeros((1, _HEADS), jnp.float32)

        p = p0_ref[...] + p1_ref[...]
        degc = jnp.clip(p[:, _C1], 1.0, None)
        d_ref[:, 0] = degc
        y = p[:, :_C1] / degc[:, None]
        sc = _lrelu(jnp.dot(y, ae_ref[...],
                            preferred_element_type=jnp.float32))
        m_ref[...] = jnp.maximum(m_ref[...], jnp.max(sc, axis=0)[None, :])
        om_blk = jnp.exp(sc - m_ref[...])
        for h in range(_HEADS):
            yhat_ref[:, _CH * h:_CH * (h + 1)] = (
                y[:, _CH * h:_CH * (h + 1)] * om_blk[:, h][:, None])
        ii = lax.broadcasted_iota(jnp.int32, (_B, _D1 - _C1), 1)
        pad = jnp.zeros((_B, _D1 - _C1), jnp.float32)
        for h in range(_HEADS):
            pad = jnp.where(ii == h, om_blk[:, h][:, None], pad)
        yhat_ref[:, _C1:] = pad

    return pl.pallas_call(
        body,
        grid=(2, nb),
        in_specs=[
            pl.BlockSpec((_B, _D1), lambda p, j: (j, 0)),
            pl.BlockSpec((_B, _D1), lambda p, j, nb=nb: (nb + j, 0)),
            pl.BlockSpec((_C1, _HEADS), lambda p, j: (0, 0)),
        ],
        out_specs=[
            pl.BlockSpec((_B, _D1), lambda p, j: (j, 0)),
            pl.BlockSpec((_B, 1), lambda p, j: (j, 0)),
        ],
        out_shape=[
            jax.ShapeDtypeStruct((_MP, _D1), jnp.float32),
            jax.ShapeDtypeStruct((_MP, 1), jnp.float32),
        ],
        scratch_shapes=[pltpu.VMEM((1, _HEADS), jnp.float32)],
    )(P1, P1, ae)


def _xform2(P2, Wo, bo):
    def body(p0_ref, p1_ref, w_ref, b_ref, o_ref):
        p = p0_ref[...] + p1_ref[...]
        cols = []
        for h in range(_HEADS):
            s = p[:, _C1 + h]
            u = p[:, _CH * h:_CH * (h + 1)]
            cols.append(_elu(u / (s + 1e-12)[:, None]))
        hfeat = jnp.concatenate(cols, axis=1)
        zt = jnp.dot(hfeat, w_ref[...],
                     preferred_element_type=jnp.float32) + b_ref[...][None, :]
        o_ref[:, :_NCLS] = zt
        o_ref[:, _NCLS:] = jnp.zeros((_B, _D2 - _NCLS), jnp.float32)

    nb = _NP // _B
    return pl.pallas_call(
        body,
        grid=(nb,),
        in_specs=[
            pl.BlockSpec((_B, _D1), lambda j: (j, 0)),
            pl.BlockSpec((_B, _D1), lambda j, nb=nb: (nb + j, 0)),
            pl.BlockSpec((_C1, _NCLS), lambda j: (0, 0)),
            pl.BlockSpec((_NCLS,), lambda j: (0,)),
        ],
        out_specs=pl.BlockSpec((_B, _D2), lambda j: (j, 0)),
        out_shape=jax.ShapeDtypeStruct((_NP, _D2), jnp.float32),
    )(P2, P2, Wo, bo)


def _edge_combined2(P3, degc, aeo):
    nb = _MP // _B

    def body(p0_ref, p1_ref, d_ref, ae_ref, yhat_ref, m_ref):
        p_idx = pl.program_id(0)
        j = pl.program_id(1)

        @pl.when(jnp.logical_and(p_idx == 0, j == 0))
        def _():
            m_ref[...] = jnp.zeros((1, 1), jnp.float32)

        p = p0_ref[...] + p1_ref[...]
        y = p[:, :_NCLS] / d_ref[...]
        sc = _lrelu(jnp.dot(y, ae_ref[...],
                            preferred_element_type=jnp.float32))
        m_ref[...] = jnp.maximum(m_ref[...], jnp.max(sc, axis=0)[None, :])
        om_col = jnp.exp(sc - m_ref[...])
        yhat_ref[:, :_NCLS] = y * om_col
        ii = lax.broadcasted_iota(jnp.int32, (_B, _D2 - _NCLS), 1)
        yhat_ref[:, _NCLS:] = jnp.where(ii == 0, om_col, 0.0)

    return pl.pallas_call(
        body,
        grid=(2, nb),
        in_specs=[
            pl.BlockSpec((_B, _D2), lambda p, j: (j, 0)),
            pl.BlockSpec((_B, _D2), lambda p, j, nb=nb: (nb + j, 0)),
            pl.BlockSpec((_B, 1), lambda p, j: (j, 0)),
            pl.BlockSpec((_NCLS, 1), lambda p, j: (0, 0)),
        ],
        out_specs=pl.BlockSpec((_B, _D2), lambda p, j: (j, 0)),
        out_shape=jax.ShapeDtypeStruct((_MP, _D2), jnp.float32),
        scratch_shapes=[pltpu.VMEM((1, 1), jnp.float32)],
    )(P3, P3, degc, aeo)


def _finalize(P4):
    def body(p0_ref, p1_ref, o_ref):
        p = p0_ref[...] + p1_ref[...]
        s = p[:, _NCLS]
        o_ref[...] = _elu(p[:, :_NCLS] / (s + 1e-12)[:, None])

    nb = _NP // _B
    return pl.pallas_call(
        body,
        grid=(nb,),
        in_specs=[
            pl.BlockSpec((_B, _D2), lambda j: (j, 0)),
            pl.BlockSpec((_B, _D2), lambda j, nb=nb: (nb + j, 0)),
        ],
        out_specs=pl.BlockSpec((_B, _NCLS), lambda j: (j, 0)),
        out_shape=jax.ShapeDtypeStruct((_NP, _NCLS), jnp.float32),
    )(P4, P4)


def kernel(X, W, b, ae, Wo, bo, aeo, v_ids, e_ids):
    Wc = W.transpose(1, 0, 2).reshape(_C1, _C1)
    bc = b.reshape(_C1)
    ae_mat = jnp.zeros((_C1, _HEADS), jnp.float32)
    for h in range(_HEADS):
        ae_mat = ae_mat.at[_CH * h:_CH * (h + 1), h].set(ae[h])
    aeo_mat = aeo[:, None]
    v1 = v_ids.reshape(_NW, _NGRP1, _GRP1, _K1)
    e1 = e_ids.reshape(_NW, _NGRP1, _GRP1, _K1)
    v2 = v_ids.reshape(_NW, _NGRP2, _GRP2, _K2)
    e2 = e_ids.reshape(_NW, _NGRP2, _GRP2, _K2)
    X_pad = jnp.pad(X, ((0, _NP - _N), (0, 0)))

    Xt_aug = _xform1(X_pad, Wc, bc)
    P1 = _seg_sum_sc(Xt_aug, v1, e1, _MP, _D1, _K1, _GRP1, _NGRP1)
    Yhat1, degc = _edge_combined1(P1, ae_mat)
    P2 = _seg_sum_sc(Yhat1, e1, v1, _NP, _D1, _K1, _GRP1, _NGRP1)

    Zt2_aug = _xform2(P2, Wo, bo)
    P3 = _seg_sum_sc(Zt2_aug, v2, e2, _MP, _D2, _K2, _GRP2, _NGRP2, nbuf=4)
    Yhat2 = _edge_combined2(P3, degc, aeo_mat)
    P4 = _seg_sum_sc(Yhat2, e2, v2, _NP, _D2, _K2, _GRP2, _NGRP2, nbuf=4)

    out = _finalize(P4)
    return out[:_N]
